# baseline (speedup 1.0000x reference)
import functools

import jax
import jax.numpy as jnp
from jax import lax
from jax.experimental import pallas as pl
from jax.experimental.pallas import tpu as pltpu
from jax.experimental.pallas import tpu_sc as plsc

_VOCAB = 1000000
_MAXLEN = 200
_EMBED = 32
_BATCH = 4096

_NC = 2
_NS = 16
_NW = _NC * _NS

_BB = _BATCH // 128
_UNITS = _MAXLEN * _BB
_PER_W = _UNITS // _NW

_VB_FULL = _VOCAB // 128
_VB_PER_W = 246
_TAIL_V = _VOCAB - _VB_FULL * 128
_TAIL_OFF = _VB_FULL * 128 * _EMBED


def _transpose_block(in_v, out_v):
    iota16 = lax.iota(jnp.int32, 16)

    def v0_body(v0, carry):
        col = jnp.full((16,), v0, jnp.int32)
        for h in range(2):
            v = plsc.load_gather(in_v, [iota16 + h * 16, col])
            out_v[pl.ds(v0 * _EMBED + h * 16, 16)] = v
        return carry

    lax.fori_loop(0, 128, v0_body, 0, unroll=4)


def _fmt_kernel(tblt_hbm, tail_hbm, out_hbm, in0, in1, out0, out1, tail_v,
                gsem0, gsem1, wsem0, wsem1):
    wid = lax.axis_index("s") * _NC + lax.axis_index("c")

    @pl.when(wid == 0)
    def _():
        pltpu.sync_copy(tail_hbm, tail_v)
        pltpu.sync_copy(tail_v, out_hbm.at[pl.ds(_TAIL_OFF, _TAIL_V * _EMBED)])

    def vb_of(j):
        return lax.min(wid * _VB_PER_W + j, _VB_FULL - 1)

    def fire_in(j, in_v, gsem):
        vb = vb_of(j)
        for eb in range(4):
            pltpu.async_copy(
                tblt_hbm.at[pl.ds(eb * 8, 8), pl.ds(vb * 128, 128)],
                in_v.at[pl.ds(eb * 8, 8), :], gsem)

    def drain_in(in_v, gsem):
        pltpu.make_async_copy(
            tblt_hbm.at[:, pl.ds(0, 128)], in_v, gsem).wait()

    def drain_out(out_v, wsem):
        pltpu.make_async_copy(
            out_v, out_hbm.at[pl.ds(0, 4096)], wsem).wait()

    fire_in(0, in0, gsem0)

    def pair_body(k, carry):
        j = 2 * k
        fire_in(j + 1, in1, gsem1)
        drain_in(in0, gsem0)

        @pl.when(k > 0)
        def _():
            drain_out(out0, wsem0)

        _transpose_block(in0, out0)
        pltpu.async_copy(out0, out_hbm.at[pl.ds(vb_of(j) * 4096, 4096)],
                         wsem0)

        fire_in(j + 2, in0, gsem0)
        drain_in(in1, gsem1)

        @pl.when(k > 0)
        def _():
            drain_out(out1, wsem1)

        _transpose_block(in1, out1)
        pltpu.async_copy(out1, out_hbm.at[pl.ds(vb_of(j + 1) * 4096, 4096)],
                         wsem1)
        return carry

    lax.fori_loop(0, _VB_PER_W // 2, pair_body, 0)
    drain_in(in0, gsem0)
    drain_out(out0, wsem0)
    drain_out(out1, wsem1)


def _emb_compute(out_hbm, u, lmin, rows_v, ro, tiles_v, posl_v, wsem):
    l = u // _BB
    bb = u % _BB
    pbase = (l - lmin) * (_EMBED * 16)
    iota16 = lax.iota(jnp.int32, 16)

    def e_body(e, carry):
        p = posl_v[pl.ds(pbase + e * 16, 16)]
        col = jnp.full((16,), e, jnp.int32)
        for g in range(8):
            row = ro + iota16 + g * 16
            v = plsc.load_gather(rows_v, [row, col])
            tiles_v[pl.ds(e * 128 + g * 16, 16)] = v + p
        return carry

    lax.fori_loop(0, _EMBED, e_body, 0)
    for eb in range(4):
        off = (l * 4 + eb) * (_BB * 1024) + bb * 1024
        pltpu.async_copy(tiles_v.at[pl.ds(eb * 1024, 1024)],
                         out_hbm.at[pl.ds(off, 1024)], wsem)


def _emb_kernel(xt_hbm, tbl_hbm, pos_hbm, out_hbm,
                idxall, rows0, rows1, tiles0, tiles1, posl_v,
                gsem0, gsem1, wsem0, wsem1):
    wid = lax.axis_index("s") * _NC + lax.axis_index("c")
    base = wid * _PER_W
    pltpu.sync_copy(xt_hbm.at[pl.ds(base * 128, _PER_W * 128)], idxall)
    lmin = lax.min(base // _BB, _MAXLEN - 8)
    pltpu.sync_copy(pos_hbm.at[pl.ds(lmin * (_EMBED * 16), 8 * _EMBED * 16)],
                    posl_v)

    def fire(j, rows_v, gsem):
        return pltpu.async_copy(
            tbl_hbm.at[idxall.at[pl.ds(j * 128, 256)]], rows_v, gsem)

    def drain_tiles(tiles_v, wsem):
        pltpu.make_async_copy(
            tiles_v, out_hbm.at[pl.ds(0, 4096)], wsem).wait()

    cp = fire(0, rows0, gsem0)
    cp.wait()

    def quad_body(k, carry):
        u = base + 4 * k
        cp1 = fire(4 * k + 2, rows1, gsem1)

        @pl.when(k > 0)
        def _():
            drain_tiles(tiles0, wsem0)

        _emb_compute(out_hbm, u, lmin, rows0, 0, tiles0, posl_v, wsem0)

        @pl.when(k > 0)
        def _():
            drain_tiles(tiles1, wsem1)

        _emb_compute(out_hbm, u + 1, lmin, rows0, 128, tiles1, posl_v, wsem1)
        cp1.wait()
        j2 = lax.min(4 * k + 4, _PER_W - 2)
        cp0 = fire(j2, rows0, gsem0)
        drain_tiles(tiles0, wsem0)
        _emb_compute(out_hbm, u + 2, lmin, rows1, 0, tiles0, posl_v, wsem0)
        drain_tiles(tiles1, wsem1)
        _emb_compute(out_hbm, u + 3, lmin, rows1, 128, tiles1, posl_v, wsem1)
        cp0.wait()
        return carry

    lax.fori_loop(0, _PER_W // 4, quad_body, 0)
    drain_tiles(tiles0, wsem0)
    drain_tiles(tiles1, wsem1)


@jax.jit
def _run(tbl, xt_flat, pos_rep):
    mesh = plsc.VectorSubcoreMesh(core_axis_name="c", subcore_axis_name="s")

    emb = functools.partial(
        pl.kernel,
        mesh=mesh,
        out_type=jax.ShapeDtypeStruct((_BATCH * _MAXLEN * _EMBED,), jnp.float32),
        scratch_types=[
            pltpu.VMEM((_PER_W * 128,), jnp.int32),
            pltpu.VMEM((256, _EMBED), jnp.float32),
            pltpu.VMEM((256, _EMBED), jnp.float32),
            pltpu.VMEM((4096,), jnp.float32),
            pltpu.VMEM((4096,), jnp.float32),
            pltpu.VMEM((8 * _EMBED * 16,), jnp.float32),
            pltpu.SemaphoreType.DMA,
            pltpu.SemaphoreType.DMA,
            pltpu.SemaphoreType.DMA,
            pltpu.SemaphoreType.DMA,
        ],
        compiler_params=pltpu.CompilerParams(
            use_tc_tiling_on_sc=False, needs_layout_passes=False),
    )(_emb_kernel)
    return emb(xt_flat, tbl, pos_rep)


def kernel(x, token_table, pos_table):
    xt_flat = jnp.swapaxes(x, 0, 1).reshape(-1).astype(jnp.int32)
    pos_rep = jnp.repeat(pos_table.reshape(-1), 16)
    out = _run(token_table, xt_flat, pos_rep)
    out5d = out.reshape(_MAXLEN, 4, _BB, 8, 128)
    return out5d.transpose(2, 4, 0, 1, 3).reshape(_BATCH, _MAXLEN, _EMBED)


# speedup vs baseline: 1.0644x; 1.0644x over previous
import functools

import jax
import jax.numpy as jnp
from jax import lax
from jax.experimental import pallas as pl
from jax.experimental.pallas import tpu as pltpu
from jax.experimental.pallas import tpu_sc as plsc

_VOCAB = 1000000
_MAXLEN = 200
_EMBED = 32
_BATCH = 4096

_PITCH = 40
_INP = 136

_NC = 2
_NS = 16
_NW = _NC * _NS

_BB = _BATCH // 128
_UNITS = _MAXLEN * _BB
_PER_W = _UNITS // _NW

_VB_FULL = _VOCAB // 128
_VB_PER_W = 246
_TAIL_V = _VOCAB - _VB_FULL * 128
_TAIL_OFF = _VB_FULL * 128 * _PITCH


def _transpose_block(in_v, out_v):
    iota16 = lax.iota(jnp.int32, 16)

    def v0_body(v0, carry):
        col = jnp.full((16,), v0, jnp.int32)
        for h in range(2):
            v = plsc.load_gather(in_v, [iota16 + h * 16, col])
            out_v[pl.ds(v0 * _PITCH + h * 16, 16)] = v
        return carry

    lax.fori_loop(0, 128, v0_body, 0, unroll=4)


def _fmt_kernel(tblt_hbm, tail_hbm, out_hbm, in0, in1, out0, out1,
                tail_v, tail_out,
                gsem0, gsem1, wsem0, wsem1):
    wid = lax.axis_index("s") * _NC + lax.axis_index("c")

    @pl.when(wid == 0)
    def _():
        pltpu.sync_copy(tail_hbm, tail_v)
        for r in range(_TAIL_V):
            for h in range(2):
                tail_out[pl.ds(r * _PITCH + h * 16, 16)] = (
                    tail_v[pl.ds(r * _EMBED + h * 16, 16)])
        pltpu.sync_copy(tail_out,
                        out_hbm.at[pl.ds(_TAIL_OFF, _TAIL_V * _PITCH)])

    def vb_of(j):
        return lax.min(wid * _VB_PER_W + j, _VB_FULL - 1)

    def fire_in(j, in_v, gsem):
        vb = vb_of(j)
        for eb in range(4):
            pltpu.async_copy(
                tblt_hbm.at[pl.ds(eb * 8, 8), pl.ds(vb * 128, 128)],
                in_v.at[pl.ds(eb * 8, 8), pl.ds(0, 128)], gsem)

    def drain_in(in_v, gsem):
        pltpu.make_async_copy(
            tblt_hbm.at[:, pl.ds(0, 128)],
            in_v.at[:, pl.ds(0, 128)], gsem).wait()

    def drain_out(out_v, wsem):
        pltpu.make_async_copy(
            out_v, out_hbm.at[pl.ds(0, 128 * _PITCH)], wsem).wait()

    fire_in(0, in0, gsem0)

    def pair_body(k, carry):
        j = 2 * k
        fire_in(j + 1, in1, gsem1)
        drain_in(in0, gsem0)

        @pl.when(k > 0)
        def _():
            drain_out(out0, wsem0)

        _transpose_block(in0, out0)
        pltpu.async_copy(
            out0, out_hbm.at[pl.ds(vb_of(j) * (128 * _PITCH), 128 * _PITCH)],
            wsem0)

        fire_in(j + 2, in0, gsem0)
        drain_in(in1, gsem1)

        @pl.when(k > 0)
        def _():
            drain_out(out1, wsem1)

        _transpose_block(in1, out1)
        pltpu.async_copy(
            out1,
            out_hbm.at[pl.ds(vb_of(j + 1) * (128 * _PITCH), 128 * _PITCH)],
            wsem1)
        return carry

    lax.fori_loop(0, _VB_PER_W // 2, pair_body, 0)
    drain_in(in0, gsem0)
    drain_out(out0, wsem0)
    drain_out(out1, wsem1)


def _emb_compute(out_hbm, u, lmin, rows_v, ro, tiles_v, posl_v, wsem):
    l = u // _BB
    bb = u % _BB
    pbase = (l - lmin) * (_EMBED * 16)
    iota16 = lax.iota(jnp.int32, 16)

    def e_body(e, carry):
        p = posl_v[pl.ds(pbase + e * 16, 16)]
        col = jnp.full((16,), e, jnp.int32)
        for g in range(8):
            row = ro + iota16 + g * 16
            v = plsc.load_gather(rows_v, [row, col])
            tiles_v[pl.ds(e * 128 + g * 16, 16)] = v + p
        return carry

    lax.fori_loop(0, _EMBED, e_body, 0)
    for eb in range(4):
        off = (l * 4 + eb) * (_BB * 1024) + bb * 1024
        pltpu.async_copy(tiles_v.at[pl.ds(eb * 1024, 1024)],
                         out_hbm.at[pl.ds(off, 1024)], wsem)


def _emb_kernel(xt_hbm, tbl_hbm, pos_hbm, out_hbm,
                idxall, rows0, rows1, tiles0, tiles1, posl_v,
                gsem0, gsem1, wsem0, wsem1):
    wid = lax.axis_index("s") * _NC + lax.axis_index("c")
    base = wid * _PER_W
    pltpu.sync_copy(xt_hbm.at[pl.ds(base * 128, _PER_W * 128)], idxall)
    lmin = lax.min(base // _BB, _MAXLEN - 8)
    pltpu.sync_copy(pos_hbm.at[pl.ds(lmin * (_EMBED * 16), 8 * _EMBED * 16)],
                    posl_v)

    def fire(j, rows_v, gsem):
        return pltpu.async_copy(
            tbl_hbm.at[idxall.at[pl.ds(j * 128, 256)]], rows_v, gsem)

    def drain_tiles(tiles_v, wsem):
        pltpu.make_async_copy(
            tiles_v, out_hbm.at[pl.ds(0, 4096)], wsem).wait()

    cp = fire(0, rows0, gsem0)
    cp.wait()

    def quad_body(k, carry):
        u = base + 4 * k
        cp1 = fire(4 * k + 2, rows1, gsem1)

        @pl.when(k > 0)
        def _():
            drain_tiles(tiles0, wsem0)

        _emb_compute(out_hbm, u, lmin, rows0, 0, tiles0, posl_v, wsem0)

        @pl.when(k > 0)
        def _():
            drain_tiles(tiles1, wsem1)

        _emb_compute(out_hbm, u + 1, lmin, rows0, 128, tiles1, posl_v, wsem1)
        cp1.wait()
        j2 = lax.min(4 * k + 4, _PER_W - 2)
        cp0 = fire(j2, rows0, gsem0)
        drain_tiles(tiles0, wsem0)
        _emb_compute(out_hbm, u + 2, lmin, rows1, 0, tiles0, posl_v, wsem0)
        drain_tiles(tiles1, wsem1)
        _emb_compute(out_hbm, u + 3, lmin, rows1, 128, tiles1, posl_v, wsem1)
        cp0.wait()
        return carry

    lax.fori_loop(0, _PER_W // 4, quad_body, 0)
    drain_tiles(tiles0, wsem0)
    drain_tiles(tiles1, wsem1)


@jax.jit
def _run(tblt, tail_lin, xt_flat, pos_rep):
    mesh = plsc.VectorSubcoreMesh(core_axis_name="c", subcore_axis_name="s")

    fmt = functools.partial(
        pl.kernel,
        mesh=mesh,
        out_type=jax.ShapeDtypeStruct((_VOCAB * _PITCH,), jnp.float32),
        scratch_types=[
            pltpu.VMEM((_EMBED, _INP), jnp.float32),
            pltpu.VMEM((_EMBED, _INP), jnp.float32),
            pltpu.VMEM((128 * _PITCH,), jnp.float32),
            pltpu.VMEM((128 * _PITCH,), jnp.float32),
            pltpu.VMEM((_TAIL_V * _EMBED,), jnp.float32),
            pltpu.VMEM((_TAIL_V * _PITCH,), jnp.float32),
            pltpu.SemaphoreType.DMA,
            pltpu.SemaphoreType.DMA,
            pltpu.SemaphoreType.DMA,
            pltpu.SemaphoreType.DMA,
        ],
        compiler_params=pltpu.CompilerParams(
            use_tc_tiling_on_sc=True, needs_layout_passes=False),
    )(_fmt_kernel)
    tbl_flat = fmt(tblt, tail_lin)
    tbl = tbl_flat.reshape(_VOCAB, _PITCH)

    emb = functools.partial(
        pl.kernel,
        mesh=mesh,
        out_type=jax.ShapeDtypeStruct((_BATCH * _MAXLEN * _EMBED,), jnp.float32),
        scratch_types=[
            pltpu.VMEM((_PER_W * 128,), jnp.int32),
            pltpu.VMEM((256, _PITCH), jnp.float32),
            pltpu.VMEM((256, _PITCH), jnp.float32),
            pltpu.VMEM((4096,), jnp.float32),
            pltpu.VMEM((4096,), jnp.float32),
            pltpu.VMEM((8 * _EMBED * 16,), jnp.float32),
            pltpu.SemaphoreType.DMA,
            pltpu.SemaphoreType.DMA,
            pltpu.SemaphoreType.DMA,
            pltpu.SemaphoreType.DMA,
        ],
        compiler_params=pltpu.CompilerParams(
            use_tc_tiling_on_sc=False, needs_layout_passes=False),
    )(_emb_kernel)
    return emb(xt_flat, tbl, pos_rep)


def kernel(x, token_table, pos_table):
    tblt = token_table.T
    tail_lin = token_table[_VB_FULL * 128:, :].reshape(-1)
    xt_flat = jnp.swapaxes(x, 0, 1).reshape(-1).astype(jnp.int32)
    pos_rep = jnp.repeat(pos_table.reshape(-1), 16)
    out = _run(tblt, tail_lin, xt_flat, pos_rep)
    out5d = out.reshape(_MAXLEN, 4, _BB, 8, 128)
    return out5d.transpose(2, 4, 0, 1, 3).reshape(_BATCH, _MAXLEN, _EMBED)


# speedup vs baseline: 1.0648x; 1.0004x over previous
import functools

import jax
import jax.numpy as jnp
from jax import lax
from jax.experimental import pallas as pl
from jax.experimental.pallas import tpu as pltpu
from jax.experimental.pallas import tpu_sc as plsc

_VOCAB = 1000000
_MAXLEN = 200
_EMBED = 32
_BATCH = 4096

_PITCH = 40
_INP = 136

_NC = 2
_NS = 16
_NW = _NC * _NS

_BB = _BATCH // 128
_UNITS = _MAXLEN * _BB
_PER_W = _UNITS // _NW

_VB_FULL = _VOCAB // 128
_VB_PER_W = 246
_TAIL_V = _VOCAB - _VB_FULL * 128
_TAIL_OFF = _VB_FULL * 128 * _PITCH


def _transpose_block(in_v, out_v):
    iota16 = lax.iota(jnp.int32, 16)

    def v0_body(v0, carry):
        col = jnp.full((16,), v0, jnp.int32)
        for h in range(2):
            v = plsc.load_gather(in_v, [iota16 + h * 16, col])
            out_v[pl.ds(v0 * _PITCH + h * 16, 16)] = v
        return carry

    lax.fori_loop(0, 128, v0_body, 0, unroll=4)


def _fmt_kernel(tblt_hbm, tail_hbm, out_hbm, in0, in1, out0, out1,
                tail_v, tail_out,
                gsem0, gsem1, wsem0, wsem1):
    wid = lax.axis_index("s") * _NC + lax.axis_index("c")

    @pl.when(wid == 0)
    def _():
        pltpu.sync_copy(tail_hbm, tail_v)
        for r in range(_TAIL_V):
            for h in range(2):
                tail_out[pl.ds(r * _PITCH + h * 16, 16)] = (
                    tail_v[pl.ds(r * _EMBED + h * 16, 16)])
        pltpu.sync_copy(tail_out,
                        out_hbm.at[pl.ds(_TAIL_OFF, _TAIL_V * _PITCH)])

    def vb_of(j):
        return lax.min(wid * _VB_PER_W + j, _VB_FULL - 1)

    def fire_in(j, in_v, gsem):
        vb = vb_of(j)
        pltpu.async_copy(
            tblt_hbm.at[:, pl.ds(vb * 128, 128)],
            in_v.at[:, pl.ds(0, 128)], gsem)

    def drain_in(in_v, gsem):
        pltpu.make_async_copy(
            tblt_hbm.at[:, pl.ds(0, 128)],
            in_v.at[:, pl.ds(0, 128)], gsem).wait()

    def drain_out(out_v, wsem):
        pltpu.make_async_copy(
            out_v, out_hbm.at[pl.ds(0, 128 * _PITCH)], wsem).wait()

    fire_in(0, in0, gsem0)

    def pair_body(k, carry):
        j = 2 * k
        fire_in(j + 1, in1, gsem1)
        drain_in(in0, gsem0)

        @pl.when(k > 0)
        def _():
            drain_out(out0, wsem0)

        _transpose_block(in0, out0)
        pltpu.async_copy(
            out0, out_hbm.at[pl.ds(vb_of(j) * (128 * _PITCH), 128 * _PITCH)],
            wsem0)

        fire_in(j + 2, in0, gsem0)
        drain_in(in1, gsem1)

        @pl.when(k > 0)
        def _():
            drain_out(out1, wsem1)

        _transpose_block(in1, out1)
        pltpu.async_copy(
            out1,
            out_hbm.at[pl.ds(vb_of(j + 1) * (128 * _PITCH), 128 * _PITCH)],
            wsem1)
        return carry

    lax.fori_loop(0, _VB_PER_W // 2, pair_body, 0)
    drain_in(in0, gsem0)
    drain_out(out0, wsem0)
    drain_out(out1, wsem1)


def _emb_compute(out_hbm, u, lmin, rows_v, ro, tiles_v, posl_v, wsem):
    l = u // _BB
    bb = u % _BB
    pbase = (l - lmin) * (_EMBED * 16)
    iota16 = lax.iota(jnp.int32, 16)

    def e_body(e, carry):
        p = posl_v[pl.ds(pbase + e * 16, 16)]
        col = jnp.full((16,), e, jnp.int32)
        for g in range(8):
            row = ro + iota16 + g * 16
            v = plsc.load_gather(rows_v, [row, col])
            tiles_v[pl.ds(e * 128 + g * 16, 16)] = v + p
        return carry

    lax.fori_loop(0, _EMBED, e_body, 0)
    for eb in range(4):
        off = (l * 4 + eb) * (_BB * 1024) + bb * 1024
        pltpu.async_copy(tiles_v.at[pl.ds(eb * 1024, 1024)],
                         out_hbm.at[pl.ds(off, 1024)], wsem)


def _emb_kernel(xt_hbm, tbl_hbm, pos_hbm, out_hbm,
                idxall, rows0, rows1, tiles0, tiles1, posl_v,
                gsem0, gsem1, wsem0, wsem1):
    wid = lax.axis_index("s") * _NC + lax.axis_index("c")
    base = wid * _PER_W
    pltpu.sync_copy(xt_hbm.at[pl.ds(base * 128, _PER_W * 128)], idxall)
    lmin = lax.min(base // _BB, _MAXLEN - 8)
    pltpu.sync_copy(pos_hbm.at[pl.ds(lmin * (_EMBED * 16), 8 * _EMBED * 16)],
                    posl_v)

    def fire(j, rows_v, gsem):
        return pltpu.async_copy(
            tbl_hbm.at[idxall.at[pl.ds(j * 128, 256)]], rows_v, gsem)

    def drain_tiles(tiles_v, wsem):
        pltpu.make_async_copy(
            tiles_v, out_hbm.at[pl.ds(0, 4096)], wsem).wait()

    cp = fire(0, rows0, gsem0)
    cp.wait()

    def quad_body(k, carry):
        u = base + 4 * k
        cp1 = fire(4 * k + 2, rows1, gsem1)

        @pl.when(k > 0)
        def _():
            drain_tiles(tiles0, wsem0)

        _emb_compute(out_hbm, u, lmin, rows0, 0, tiles0, posl_v, wsem0)

        @pl.when(k > 0)
        def _():
            drain_tiles(tiles1, wsem1)

        _emb_compute(out_hbm, u + 1, lmin, rows0, 128, tiles1, posl_v, wsem1)
        cp1.wait()
        j2 = lax.min(4 * k + 4, _PER_W - 2)
        cp0 = fire(j2, rows0, gsem0)
        drain_tiles(tiles0, wsem0)
        _emb_compute(out_hbm, u + 2, lmin, rows1, 0, tiles0, posl_v, wsem0)
        drain_tiles(tiles1, wsem1)
        _emb_compute(out_hbm, u + 3, lmin, rows1, 128, tiles1, posl_v, wsem1)
        cp0.wait()
        return carry

    lax.fori_loop(0, _PER_W // 4, quad_body, 0)
    drain_tiles(tiles0, wsem0)
    drain_tiles(tiles1, wsem1)


@jax.jit
def _run(tblt, tail_lin, xt_flat, pos_rep):
    mesh = plsc.VectorSubcoreMesh(core_axis_name="c", subcore_axis_name="s")

    fmt = functools.partial(
        pl.kernel,
        mesh=mesh,
        out_type=jax.ShapeDtypeStruct((_VOCAB * _PITCH,), jnp.float32),
        scratch_types=[
            pltpu.VMEM((_EMBED, _INP), jnp.float32),
            pltpu.VMEM((_EMBED, _INP), jnp.float32),
            pltpu.VMEM((128 * _PITCH,), jnp.float32),
            pltpu.VMEM((128 * _PITCH,), jnp.float32),
            pltpu.VMEM((_TAIL_V * _EMBED,), jnp.float32),
            pltpu.VMEM((_TAIL_V * _PITCH,), jnp.float32),
            pltpu.SemaphoreType.DMA,
            pltpu.SemaphoreType.DMA,
            pltpu.SemaphoreType.DMA,
            pltpu.SemaphoreType.DMA,
        ],
        compiler_params=pltpu.CompilerParams(
            use_tc_tiling_on_sc=True, needs_layout_passes=False),
    )(_fmt_kernel)
    tbl_flat = fmt(tblt, tail_lin)
    tbl = tbl_flat.reshape(_VOCAB, _PITCH)

    emb = functools.partial(
        pl.kernel,
        mesh=mesh,
        out_type=jax.ShapeDtypeStruct((_BATCH * _MAXLEN * _EMBED,), jnp.float32),
        scratch_types=[
            pltpu.VMEM((_PER_W * 128,), jnp.int32),
            pltpu.VMEM((256, _PITCH), jnp.float32),
            pltpu.VMEM((256, _PITCH), jnp.float32),
            pltpu.VMEM((4096,), jnp.float32),
            pltpu.VMEM((4096,), jnp.float32),
            pltpu.VMEM((8 * _EMBED * 16,), jnp.float32),
            pltpu.SemaphoreType.DMA,
            pltpu.SemaphoreType.DMA,
            pltpu.SemaphoreType.DMA,
            pltpu.SemaphoreType.DMA,
        ],
        compiler_params=pltpu.CompilerParams(
            use_tc_tiling_on_sc=False, needs_layout_passes=False),
    )(_emb_kernel)
    return emb(xt_flat, tbl, pos_rep)


def kernel(x, token_table, pos_table):
    tblt = token_table.T
    tail_lin = token_table[_VB_FULL * 128:, :].reshape(-1)
    xt_flat = jnp.swapaxes(x, 0, 1).reshape(-1).astype(jnp.int32)
    pos_rep = jnp.repeat(pos_table.reshape(-1), 16)
    out = _run(tblt, tail_lin, xt_flat, pos_rep)
    out5d = out.reshape(_MAXLEN, 4, _BB, 8, 128)
    return out5d.transpose(2, 4, 0, 1, 3).reshape(_BATCH, _MAXLEN, _EMBED)
